# baseline (speedup 1.0000x reference)
import math

import jax
import jax.numpy as jnp
from jax import lax
from jax.experimental import pallas as pl
from jax.experimental.pallas import tpu as pltpu
from jax.experimental.pallas import tpu_sc as plsc

_NUM_CORES = 2
_NUM_SUBCORES = 16
_NUM_WORKERS = _NUM_CORES * _NUM_SUBCORES

_IDX_W = 128
_K = 2
_CHUNK = _K * _IDX_W
_NBUF = 4

_TW = 512


def _tpose_body(scale, lt_ref, out_ref):
  blk = lt_ref[...]
  t = blk.T * scale
  out_ref[...] = jnp.concatenate([t, t], axis=1)


def _scaled_rowmajor_table(lut):
  v, d = lut.shape
  lt = lut.T
  grid = (v + _TW - 1) // _TW
  return pl.pallas_call(
      lambda lt_ref, out_ref: _tpose_body(math.sqrt(d), lt_ref, out_ref),
      out_shape=jax.ShapeDtypeStruct((v, 2 * d), jnp.float32),
      grid=(grid,),
      in_specs=[pl.BlockSpec((d, _TW), lambda c: (0, c))],
      out_specs=pl.BlockSpec((_TW, 2 * d), lambda c: (c, 0)),
  )(lt)


def _gather_body(nchunks, x_hbm, tab_hbm, out_hbm,
                 ib0, ib1, ib2, ib3, gb0, gb1, gb2, gb3,
                 isem, gsem0, gsem1, gsem2, gsem3,
                 osem0, osem1, osem2, osem3):
  rows_per_w = nchunks * _CHUNK
  irows_per_w = nchunks * _K

  wid = lax.axis_index("s") * _NUM_CORES + lax.axis_index("c")
  irow0 = wid * irows_per_w
  row0 = wid * rows_per_w

  ibufs = (ib0, ib1, ib2, ib3)
  gbufs = (gb0, gb1, gb2, gb3)
  gsems = (gsem0, gsem1, gsem2, gsem3)
  osems = (osem0, osem1, osem2, osem3)

  def idx_load(g, b):
    pltpu.async_copy(
        x_hbm.at[pl.ds(irow0 + g * _K, _K)], ibufs[b], isem).wait()
    for r in range(_K):
      for k in range(_IDX_W // 16):
        sl = pl.ds(k * 16, 16)
        ibufs[b][r, sl] = ibufs[b][r, sl] * 2

  def gather_start(b):
    for j in range(_K):
      pltpu.async_copy(
          tab_hbm.at[ibufs[b].at[j]],
          gbufs[b].at[pl.ds(j * _IDX_W, _IDX_W)],
          gsems[b])

  def gather_wait(b):
    for j in range(_K):
      pltpu.make_async_copy(
          tab_hbm.at[ibufs[b].at[j]],
          gbufs[b].at[pl.ds(j * _IDX_W, _IDX_W)],
          gsems[b]).wait()

  def out_start(g, b):
    pltpu.async_copy(
        gbufs[b], out_hbm.at[pl.ds(row0 + g * _CHUNK, _CHUNK)], osems[b])

  def out_wait(g, b):
    pltpu.make_async_copy(
        gbufs[b], out_hbm.at[pl.ds(row0 + g * _CHUNK, _CHUNK)],
        osems[b]).wait()

  for b in range(2):
    idx_load(b, b)
    gather_start(b)

  @pl.loop(0, nchunks, step=_NBUF)
  def _steady(g0):
    for b in range(_NBUF):
      g = g0 + b
      gather_wait(b)
      out_start(g, b)

      b2 = (b + 2) % _NBUF

      @pl.when(g >= 2)
      def _():
        out_wait(g - 2, b2)

      @pl.when(g + 2 < nchunks)
      def _():
        idx_load(g + 2, b2)
        gather_start(b2)

  for g in (nchunks - 2, nchunks - 1):
    out_wait(g, g % _NBUF)


def _sc_gather(x2d, table, nchunks, n, dim):
  mesh = plsc.VectorSubcoreMesh(
      core_axis_name="c", subcore_axis_name="s",
      num_cores=_NUM_CORES, num_subcores=_NUM_SUBCORES)
  run = pl.kernel(
      lambda *refs: _gather_body(nchunks, *refs),
      out_type=jax.ShapeDtypeStruct((n, dim), jnp.float32),
      mesh=mesh,
      scratch_types=(
          [pltpu.VMEM((_K, _IDX_W), jnp.int32) for _ in range(_NBUF)]
          + [pltpu.VMEM((_CHUNK, dim), jnp.float32) for _ in range(_NBUF)]
          + [pltpu.SemaphoreType.DMA] * (1 + 2 * _NBUF)
      ),
      compiler_params=pltpu.CompilerParams(use_tc_tiling_on_sc=False),
      name="sc_embedding_lookup",
  )
  return run(x2d, table)


def kernel(x, lut):
  batch_shape = x.shape
  vocab, dim = lut.shape
  n = x.size
  assert n % (_NUM_WORKERS * _CHUNK) == 0
  assert (vocab * dim) % 128 == 0
  nchunks = n // (_NUM_WORKERS * _CHUNK)

  packed = _scaled_rowmajor_table(lut)
  table = packed.reshape(2 * vocab, dim)
  x2d = x.reshape(-1).astype(jnp.int32).reshape(n // _IDX_W, _IDX_W)
  out = _sc_gather(x2d, table, nchunks, n, dim)
  return out.reshape(*batch_shape, dim)


# speedup vs baseline: 1.0013x; 1.0013x over previous
import math

import jax
import jax.numpy as jnp
from jax import lax
from jax.experimental import pallas as pl
from jax.experimental.pallas import tpu as pltpu
from jax.experimental.pallas import tpu_sc as plsc

_NUM_CORES = 2
_NUM_SUBCORES = 16
_NUM_WORKERS = _NUM_CORES * _NUM_SUBCORES
_LANES = 16

_IDX_W = 128
_U = 2
_STEP_ROWS = _U * _IDX_W


def _gather_body(nsteps, dim, x_hbm, tab_hbm, out_hbm,
                 ib0, ib1, gb0, gb1, tb0, tb1,
                 isem, gsem0, gsem1, osem0, osem1):
  scale = dim ** 0.5
  ndt = dim // 8
  blk = _U * 8 * _IDX_W
  steps_per_slab = _IDX_W // _U

  wid = lax.axis_index("s") * _NUM_CORES + lax.axis_index("c")
  step0 = wid * nsteps

  ibufs = (ib0, ib1)
  gbufs = (gb0, gb1)
  tbufs = (tb0, tb1)
  gsems = (gsem0, gsem1)
  osems = (osem0, osem1)

  iot = lax.iota(jnp.int32, _LANES)

  def idx_rows(u):
    s = u // steps_per_slab
    bt0 = (u % steps_per_slab) * _U
    return s * _IDX_W + bt0

  def out_off(u):
    s = u // steps_per_slab
    bt0 = (u % steps_per_slab) * _U
    return (s * ndt * _IDX_W + bt0) * (8 * _IDX_W)

  def idx_load(u, p):
    pltpu.async_copy(x_hbm.at[pl.ds(idx_rows(u), _U)], ibufs[p], isem).wait()

  def gather_start(p):
    for j in range(_U):
      pltpu.async_copy(
          tab_hbm.at[ibufs[p].at[j]],
          gbufs[p].at[pl.ds(j * _IDX_W, _IDX_W)],
          gsems[p])

  def gather_wait(p):
    for j in range(_U):
      pltpu.make_async_copy(
          tab_hbm.at[ibufs[p].at[j]],
          gbufs[p].at[pl.ds(j * _IDX_W, _IDX_W)],
          gsems[p]).wait()

  def out_start(u, p):
    base = out_off(u)
    for dt in range(8):
      pltpu.async_copy(
          tbufs[p].at[pl.ds(dt * blk, blk)],
          out_hbm.at[pl.ds(base + dt * _IDX_W * (8 * _IDX_W), blk)],
          osems[p])

  def out_wait(u, p):
    base = out_off(u)
    for dt in range(8):
      pltpu.make_async_copy(
          tbufs[p].at[pl.ds(dt * blk, blk)],
          out_hbm.at[pl.ds(base + dt * _IDX_W * (8 * _IDX_W), blk)],
          osems[p]).wait()

  def transpose_scale(p):
    gbuf = gbufs[p]
    tbuf = tbufs[p]

    @plsc.parallel_loop(0, 8 * _U)
    def _outer(m):
      dt = m // _U
      btl = m % _U
      tb = dt * blk + btl * (8 * _IDX_W)
      gb = btl * _IDX_W
      for di in range(8):
        cols = jnp.full((_LANES,), 8 * dt + di, jnp.int32)
        for k in range(_IDX_W // _LANES):
          rows = iot + (gb + k * _LANES)
          v = plsc.load_gather(gbuf, [rows, cols])
          tbuf[pl.ds(tb + di * _IDX_W + k * _LANES, _LANES)] = v * scale

  for p in range(2):
    idx_load(step0 + p, p)
    gather_start(p)

  @pl.loop(0, nsteps, step=2)
  def _steady(i0):
    for p in range(2):
      i = i0 + p
      u = step0 + i
      gather_wait(p)

      @pl.when(i >= 2)
      def _():
        out_wait(u - 2, p)

      transpose_scale(p)

      @pl.when(i + 2 < nsteps)
      def _():
        idx_load(u + 2, p)
        gather_start(p)

      out_start(u, p)

  for i in (nsteps - 2, nsteps - 1):
    out_wait(step0 + i, i % 2)


def kernel(x, lut):
  batch, seq = x.shape
  vocab, dim = lut.shape
  n = x.size
  assert batch % (_IDX_W * _U) == 0 and dim % 8 == 0
  nsteps_total = n // _STEP_ROWS
  assert nsteps_total % _NUM_WORKERS == 0
  nsteps = nsteps_total // _NUM_WORKERS

  xs = jnp.transpose(x).astype(jnp.int32).reshape(seq * (batch // _IDX_W),
                                                  _IDX_W)

  mesh = plsc.VectorSubcoreMesh(
      core_axis_name="c", subcore_axis_name="s",
      num_cores=_NUM_CORES, num_subcores=_NUM_SUBCORES)
  gflat = _STEP_ROWS * dim
  run = pl.kernel(
      lambda *refs: _gather_body(nsteps, dim, *refs),
      out_type=jax.ShapeDtypeStruct((n * dim,), jnp.float32),
      mesh=mesh,
      scratch_types=(
          [pltpu.VMEM((_U, _IDX_W), jnp.int32) for _ in range(2)]
          + [pltpu.VMEM((_STEP_ROWS, dim), jnp.float32) for _ in range(2)]
          + [pltpu.VMEM((gflat,), jnp.float32) for _ in range(2)]
          + [pltpu.SemaphoreType.DMA] * 5
      ),
      compiler_params=pltpu.CompilerParams(use_tc_tiling_on_sc=False,
                                           needs_layout_passes=False),
      name="sc_embedding_lookup",
  )
  out5 = run(xs, lut)
  out = out5.reshape(seq, dim // 8, batch // _IDX_W, 8, _IDX_W)
  return out.transpose(2, 4, 0, 1, 3).reshape(batch, seq, dim)


# speedup vs baseline: 1.5407x; 1.5388x over previous
import math

import jax
import jax.numpy as jnp
from jax import lax
from jax.experimental import pallas as pl
from jax.experimental.pallas import tpu as pltpu
from jax.experimental.pallas import tpu_sc as plsc

_NUM_CORES = 2
_NUM_SUBCORES = 16
_NUM_WORKERS = _NUM_CORES * _NUM_SUBCORES
_LANES = 16

_IDX_W = 128
_U = 2
_STEP_ROWS = _U * _IDX_W


def _gather_body(nsteps, dim, x_hbm, tab_hbm, out_hbm,
                 ib0, ib1, gb0, gb1, g20, g21, tb0, tb1,
                 isem, gsem0, gsem1, osem0, osem1):
  scale = dim ** 0.5
  ndt = dim // 8
  blk = _U * 8 * _IDX_W
  steps_per_slab = _IDX_W // _U

  wid = lax.axis_index("s") * _NUM_CORES + lax.axis_index("c")
  step0 = wid * nsteps

  ibufs = (ib0, ib1)
  gbufs = (gb0, gb1)
  g2bufs = (g20, g21)
  tbufs = (tb0, tb1)
  gsems = (gsem0, gsem1)
  osems = (osem0, osem1)

  pitch = dim + 1
  iotp = lax.iota(jnp.int32, _LANES) * pitch

  def idx_rows(u):
    s = u // steps_per_slab
    bt0 = (u % steps_per_slab) * _U
    return s * _IDX_W + bt0

  def out_off(u):
    s = u // steps_per_slab
    bt0 = (u % steps_per_slab) * _U
    return (s * ndt * _IDX_W + bt0) * (8 * _IDX_W)

  def idx_load(u, p):
    pltpu.async_copy(x_hbm.at[pl.ds(idx_rows(u), _U)], ibufs[p], isem).wait()

  def gather_start(p):
    for j in range(_U):
      pltpu.async_copy(
          tab_hbm.at[ibufs[p].at[j]],
          gbufs[p].at[pl.ds(j * _IDX_W, _IDX_W)],
          gsems[p])

  def gather_wait(p):
    for j in range(_U):
      pltpu.make_async_copy(
          tab_hbm.at[ibufs[p].at[j]],
          gbufs[p].at[pl.ds(j * _IDX_W, _IDX_W)],
          gsems[p]).wait()

  def out_start(u, p):
    base = out_off(u)
    for dt in range(8):
      pltpu.async_copy(
          tbufs[p].at[pl.ds(dt * blk, blk)],
          out_hbm.at[pl.ds(base + dt * _IDX_W * (8 * _IDX_W), blk)],
          osems[p])

  def out_wait(u, p):
    base = out_off(u)
    for dt in range(8):
      pltpu.make_async_copy(
          tbufs[p].at[pl.ds(dt * blk, blk)],
          out_hbm.at[pl.ds(base + dt * _IDX_W * (8 * _IDX_W), blk)],
          osems[p]).wait()

  def transpose_scale(p):
    gbuf = gbufs[p]
    g2 = g2bufs[p]
    tbuf = tbufs[p]

    @plsc.parallel_loop(0, _STEP_ROWS, unroll=4)
    def _repitch(r):
      for q in range(dim // _LANES):
        g2[pl.ds(r * pitch + q * _LANES, _LANES)] = (
            gbuf[r, pl.ds(q * _LANES, _LANES)])

    @plsc.parallel_loop(0, 8 * _U)
    def _outer(m):
      dt = m // _U
      btl = m % _U
      tb = dt * blk + btl * (8 * _IDX_W)
      gb = btl * _IDX_W * pitch
      for di in range(8):
        col = 8 * dt + di
        for k in range(_IDX_W // _LANES):
          addr = iotp + (gb + k * _LANES * pitch + col)
          v = plsc.load_gather(g2, [addr])
          tbuf[pl.ds(tb + di * _IDX_W + k * _LANES, _LANES)] = v * scale

  for p in range(2):
    idx_load(step0 + p, p)
    gather_start(p)

  @pl.loop(0, nsteps, step=2)
  def _steady(i0):
    for p in range(2):
      i = i0 + p
      u = step0 + i
      gather_wait(p)

      @pl.when(i >= 2)
      def _():
        out_wait(u - 2, p)

      transpose_scale(p)

      @pl.when(i + 2 < nsteps)
      def _():
        idx_load(u + 2, p)
        gather_start(p)

      out_start(u, p)

  for i in (nsteps - 2, nsteps - 1):
    out_wait(step0 + i, i % 2)


def kernel(x, lut):
  batch, seq = x.shape
  vocab, dim = lut.shape
  n = x.size
  assert batch % (_IDX_W * _U) == 0 and dim % 8 == 0
  nsteps_total = n // _STEP_ROWS
  assert nsteps_total % _NUM_WORKERS == 0
  nsteps = nsteps_total // _NUM_WORKERS

  xs = jnp.transpose(x).astype(jnp.int32).reshape(seq * (batch // _IDX_W),
                                                  _IDX_W)

  mesh = plsc.VectorSubcoreMesh(
      core_axis_name="c", subcore_axis_name="s",
      num_cores=_NUM_CORES, num_subcores=_NUM_SUBCORES)
  gflat = _STEP_ROWS * dim
  run = pl.kernel(
      lambda *refs: _gather_body(nsteps, dim, *refs),
      out_type=jax.ShapeDtypeStruct((n * dim,), jnp.float32),
      mesh=mesh,
      scratch_types=(
          [pltpu.VMEM((_U, _IDX_W), jnp.int32) for _ in range(2)]
          + [pltpu.VMEM((_STEP_ROWS, dim), jnp.float32) for _ in range(2)]
          + [pltpu.VMEM((_STEP_ROWS * (dim + 1),), jnp.float32)
             for _ in range(2)]
          + [pltpu.VMEM((gflat,), jnp.float32) for _ in range(2)]
          + [pltpu.SemaphoreType.DMA] * 5
      ),
      compiler_params=pltpu.CompilerParams(use_tc_tiling_on_sc=False,
                                           needs_layout_passes=False),
      name="sc_embedding_lookup",
  )
  out5 = run(xs, lut)
  out = out5.reshape(seq, dim // 8, batch // _IDX_W, 8, _IDX_W)
  return out.transpose(2, 4, 0, 1, 3).reshape(batch, seq, dim)
